# baseline (speedup 1.0000x reference)
import functools

import jax
import jax.numpy as jnp
from jax import lax
from jax.experimental import pallas as pl
from jax.experimental.pallas import tpu as pltpu
from jax.experimental.pallas import tpu_sc as plsc

B, T = 4, 2048
D, K = 1024, 512
N = B * T
NC, NS = 2, 16
NW = NC * NS
CH = 4
NCROWS = N // CH
GPW = NCROWS // NW
GCHUNK = 32
NGCH = GPW // GCHUNK
TPW = NCROWS // NW
WPR = T // TPW
ROW_BLOCK = 512
N_BLOCKS = NCROWS // ROW_BLOCK

_f32 = jnp.float32
_i32 = jnp.int32


def _sc_embed_body(tok_hbm, emb_hbm, out_hbm, idx_v, buf_a, buf_b,
                   gsem_a, gsem_b, wsem_a, wsem_b):
    c = lax.axis_index("c")
    s = lax.axis_index("s")
    wid = s * NC + c
    base = wid * GPW
    pltpu.sync_copy(tok_hbm.at[pl.ds(base, GPW)], idx_v)
    bufs = (buf_a, buf_b)
    gsems = (gsem_a, gsem_b)
    wsems = (wsem_a, wsem_b)
    gathers = [None] * NGCH
    writes = [None] * NGCH
    gathers[0] = pltpu.async_copy(
        emb_hbm.at[idx_v.at[pl.ds(0, GCHUNK)]], bufs[0], gsems[0])
    for ch in range(NGCH):
        cur = ch % 2
        gathers[ch].wait()
        writes[ch] = pltpu.async_copy(
            bufs[cur], out_hbm.at[pl.ds(base + ch * GCHUNK, GCHUNK)],
            wsems[cur])
        if ch + 1 < NGCH:
            nxt = (ch + 1) % 2
            if ch >= 1:
                writes[ch - 1].wait()
            gathers[ch + 1] = pltpu.async_copy(
                emb_hbm.at[idx_v.at[pl.ds((ch + 1) * GCHUNK, GCHUNK)]],
                bufs[nxt], gsems[nxt])
    writes[NGCH - 1].wait()


def _sc_embed_gather(tok, embedding):
    mesh = plsc.VectorSubcoreMesh(core_axis_name="c", subcore_axis_name="s")
    f = functools.partial(
        pl.kernel,
        mesh=mesh,
        out_type=jax.ShapeDtypeStruct((NCROWS, D), _f32),
        scratch_types=[
            pltpu.VMEM((GPW,), _i32),
            pltpu.VMEM((GCHUNK, D), _f32),
            pltpu.VMEM((GCHUNK, D), _f32),
            pltpu.SemaphoreType.DMA,
            pltpu.SemaphoreType.DMA,
            pltpu.SemaphoreType.DMA,
            pltpu.SemaphoreType.DMA,
        ],
    )(_sc_embed_body)
    return f(tok, embedding)


def _prologue_body(cb_ref, cbn_ref, tbl_ref, g_ref):
    cb = cb_ref[...]
    n2 = jnp.sum(cb * cb, axis=1, keepdims=True)
    n = jnp.sqrt(n2)
    mcol = jnp.maximum(n, 1e-12)
    cbn_ref[...] = (cb / mcol).astype(jnp.bfloat16)
    pn = cb / (n + 1e-6)
    p2 = pn * pn
    s2 = jnp.sum(p2, axis=1, keepdims=True)
    s4 = jnp.sum(p2 * p2, axis=1, keepdims=True)
    s6 = jnp.sum(p2 * p2 * p2, axis=1, keepdims=True)
    z = jnp.zeros_like(s2)
    tbl_ref[...] = jnp.concatenate(
        [s2.T, s4.T, s6.T, n2.T, mcol.T, z.T, z.T, z.T], axis=0)
    g_ref[...] = lax.dot_general(
        pn, pn, (((1,), (1,)), ((), ())),
        preferred_element_type=_f32, precision=lax.Precision.HIGHEST)


def _prologue(codebook):
    return pl.pallas_call(
        _prologue_body,
        out_shape=[
            jax.ShapeDtypeStruct((K, D), jnp.bfloat16),
            jax.ShapeDtypeStruct((8, K), _f32),
            jax.ShapeDtypeStruct((K, K), _f32),
        ],
    )(codebook)


def _main_body(ze_ref, cbn_ref, k_ref, rmm_ref, cs_ref, rn_ref):
    i = pl.program_id(0)
    ze = ze_ref[...]
    rown2 = jnp.sum(ze * ze, axis=1, keepdims=True)
    m_row = jnp.maximum(jnp.sqrt(rown2), 1e-12)
    fn = (ze / m_row).astype(jnp.bfloat16)
    sim = lax.dot_general(
        fn, cbn_ref[...], (((1,), (1,)), ((), ())),
        preferred_element_type=_f32)
    rowmax = jnp.max(sim, axis=1, keepdims=True)
    io = lax.broadcasted_iota(_i32, sim.shape, 1)
    kk = jnp.min(jnp.where(sim == rowmax, io, K), axis=1)
    k_ref[...] = kk
    rmm_ref[...] = (rowmax * m_row)[:, 0]
    p = jnp.exp(sim - rowmax)
    probs = p * (1.0 / jnp.sum(p, axis=1, keepdims=True))

    @pl.when(i == 0)
    def _init():
        cs_ref[...] = jnp.zeros_like(cs_ref)
        rn_ref[...] = jnp.zeros_like(rn_ref)

    cs_ref[...] += jnp.sum(probs, axis=0, keepdims=True)
    rn_ref[...] += jnp.reshape(jnp.sum(rown2), (1, 1))


def _main(ze, cbn):
    return pl.pallas_call(
        _main_body,
        grid=(N_BLOCKS,),
        in_specs=[
            pl.BlockSpec((ROW_BLOCK, D), lambda i: (i, 0)),
            pl.BlockSpec((K, D), lambda i: (0, 0)),
        ],
        out_specs=[
            pl.BlockSpec((ROW_BLOCK,), lambda i: (i,)),
            pl.BlockSpec((ROW_BLOCK,), lambda i: (i,)),
            pl.BlockSpec((1, K), lambda i: (0, 0)),
            pl.BlockSpec((1, 1), lambda i: (0, 0)),
        ],
        out_shape=[
            jax.ShapeDtypeStruct((NCROWS,), _i32),
            jax.ShapeDtypeStruct((NCROWS,), _f32),
            jax.ShapeDtypeStruct((1, K), _f32),
            jax.ShapeDtypeStruct((1, 1), _f32),
        ],
    )(ze, cbn)


def _sc_tab_body(k_hbm, tbl_hbm, g_hbm,
                 s2o, s4o, s6o, c2o, mco, gno,
                 krow, tblv, gidx, s2b, s4b, s6b, c2b, mcb, gob, sem):
    c = lax.axis_index("c")
    s = lax.axis_index("s")
    wid = s * NC + c
    bl = wid // WPR
    t0 = (wid % WPR) * TPW
    pltpu.sync_copy(k_hbm.at[pl.ds(bl * T, T)], krow)
    pltpu.sync_copy(tbl_hbm, tblv)
    for j in range(TPW // 16):
        t = t0 + j * 16 + lax.iota(_i32, 16)
        iself = plsc.load_gather(krow, [t])
        inext = plsc.load_gather(krow, [lax.rem(t + 1, T)])
        sl = pl.ds(j * 16, 16)
        gidx[sl] = iself * K + inext
        s2b[sl] = plsc.load_gather(tblv, [iself])
        s4b[sl] = plsc.load_gather(tblv, [iself + K])
        s6b[sl] = plsc.load_gather(tblv, [iself + 2 * K])
        c2b[sl] = plsc.load_gather(tblv, [iself + 3 * K])
        mcb[sl] = plsc.load_gather(tblv, [iself + 4 * K])
    pltpu.async_copy(g_hbm.at[gidx], gob, sem).wait()
    base = wid * TPW
    pltpu.sync_copy(s2b, s2o.at[pl.ds(base, TPW)])
    pltpu.sync_copy(s4b, s4o.at[pl.ds(base, TPW)])
    pltpu.sync_copy(s6b, s6o.at[pl.ds(base, TPW)])
    pltpu.sync_copy(c2b, c2o.at[pl.ds(base, TPW)])
    pltpu.sync_copy(mcb, mco.at[pl.ds(base, TPW)])
    pltpu.sync_copy(gob, gno.at[pl.ds(base, TPW)])


def _sc_tab_gather(kchunk, tblflat, gflat):
    mesh = plsc.VectorSubcoreMesh(core_axis_name="c", subcore_axis_name="s")
    vecN = jax.ShapeDtypeStruct((NCROWS,), _f32)
    f = functools.partial(
        pl.kernel,
        mesh=mesh,
        compiler_params=pltpu.CompilerParams(needs_layout_passes=False),
        out_type=[vecN] * 6,
        scratch_types=[
            pltpu.VMEM((T,), _i32),
            pltpu.VMEM((8 * K,), _f32),
            pltpu.VMEM((TPW,), _i32),
            pltpu.VMEM((TPW,), _f32),
            pltpu.VMEM((TPW,), _f32),
            pltpu.VMEM((TPW,), _f32),
            pltpu.VMEM((TPW,), _f32),
            pltpu.VMEM((TPW,), _f32),
            pltpu.VMEM((TPW,), _f32),
            pltpu.SemaphoreType.DMA,
        ],
    )(_sc_tab_body)
    return f(kchunk, tblflat, gflat)


def _final_body(*refs):
    (s2c, s4c, s6c, c2c, mcc, gnc, rmc) = (
        refs[i * CH:(i + 1) * CH] for i in range(7))
    csc = refs[7 * CH:8 * CH]
    rnc = refs[8 * CH:9 * CH]
    lm2, lg4, lg6, mu_ = refs[9 * CH:9 * CH + 4]
    sd_ref, sm_ref, cl_ref, pp_ref = refs[9 * CH + 4:]
    m2 = jnp.exp(lm2[0, 0])
    g4 = jnp.exp(lg4[0, 0])
    g6 = jnp.exp(lg6[0, 0])
    emu = jnp.exp(mu_[0, 0])
    enmu = jnp.exp(-mu_[0, 0])
    cat = lambda rs: jnp.concatenate([r[...] for r in rs])
    s2 = cat(s2c)
    s4 = cat(s4c)
    s6 = cat(s6c)
    c2 = cat(c2c)
    mc = cat(mcc)
    gn = cat(gnc)
    rm = cat(rmc)
    tmod = lax.rem(lax.broadcasted_iota(_i32, (N,), 0), T)
    gp = jnp.where(tmod == 0, jnp.roll(gn, 1 - T), jnp.roll(gn, 1))
    chem = -0.5 * (emu * gn + enmu * gp)
    mass = (-0.5 * m2) * s2
    p4 = (-g4 / 24.0) * s4
    p6 = (-g6 / 720.0) * s6
    s2n = jnp.roll(s2, -1)
    kin = 0.5 * (s2 + s2n) - gn
    sd = mass + p4 + p6 + chem + jnp.where(tmod < T - 1, kin, 0.0)
    sd_ref[...] = sd
    sm_ref[...] = jnp.reshape(
        jnp.stack([jnp.sum(sd[b * T:(b + 1) * T]) for b in range(B)]), (1, B))
    rntot = sum(r[0, 0] for r in rnc)
    commit = (rntot - 2.0 * jnp.sum(rm * mc)
              + jnp.sum(c2)) * (1.0 / (N * D))
    cl_ref[...] = jnp.reshape(commit, (1, 1))
    avg = sum(c[...] for c in csc) * (1.0 / N)
    pp = jnp.exp(-jnp.sum(avg * jnp.log(avg + 1e-10)))
    pp_ref[...] = jnp.reshape(pp, (1, 1))


def _final(args):
    return pl.pallas_call(
        _final_body,
        out_shape=[
            jax.ShapeDtypeStruct((N,), _f32),
            jax.ShapeDtypeStruct((1, B), _f32),
            jax.ShapeDtypeStruct((1, 1), _f32),
            jax.ShapeDtypeStruct((1, 1), _f32),
        ],
    )(*args)


def kernel(token_ids, embedding, codebook, log_m2, log_g4, log_g6, mu):
    tok = token_ids.reshape(-1).astype(_i32)
    zes = [_sc_embed_gather(tok[c * NCROWS:(c + 1) * NCROWS], embedding)
           for c in range(CH)]
    cbn, tbl, g = _prologue(codebook)
    tblflat = tbl.reshape(-1)
    gflat = g.reshape(-1)
    mains = [_main(ze, cbn) for ze in zes]
    tabs = [_sc_tab_gather(m[0], tblflat, gflat) for m in mains]
    p11 = lambda x: jnp.reshape(x.astype(_f32), (1, 1))
    args = []
    for i in range(6):
        args += [t[i] for t in tabs]
    args += [m[1] for m in mains]
    args += [m[2] for m in mains]
    args += [m[3] for m in mains]
    args += [p11(log_m2), p11(log_g4), p11(log_g6), p11(mu)]
    sd, sm, cl, pp = _final(args)
    quanta = jnp.concatenate([m[0] for m in mains]).reshape(B, T)
    return (sm.reshape(B), sd.reshape(B, T), quanta, cl.reshape(()),
            pp.reshape(()))


# speedup vs baseline: 1.1829x; 1.1829x over previous
import functools

import jax
import jax.numpy as jnp
from jax import lax
from jax.experimental import pallas as pl
from jax.experimental.pallas import tpu as pltpu
from jax.experimental.pallas import tpu_sc as plsc

B, T = 4, 2048
D, K = 1024, 512
N = B * T
NC, NS = 2, 16
NW = NC * NS
CH = 2
NCROWS = N // CH
GPW = NCROWS // NW
GCHUNK = 32
NGCH = GPW // GCHUNK
TPW = NCROWS // NW
WPR = T // TPW
ROW_BLOCK = 1024
N_BLOCKS = NCROWS // ROW_BLOCK

_f32 = jnp.float32
_i32 = jnp.int32


def _sc_embed_body(tok_hbm, emb_hbm, out_hbm, idx_v, buf_a, buf_b,
                   gsem_a, gsem_b, wsem_a, wsem_b):
    c = lax.axis_index("c")
    s = lax.axis_index("s")
    wid = s * NC + c
    base = wid * GPW
    pltpu.sync_copy(tok_hbm.at[pl.ds(base, GPW)], idx_v)
    bufs = (buf_a, buf_b)
    gsems = (gsem_a, gsem_b)
    wsems = (wsem_a, wsem_b)
    gathers = [None] * NGCH
    writes = [None] * NGCH
    gathers[0] = pltpu.async_copy(
        emb_hbm.at[idx_v.at[pl.ds(0, GCHUNK)]], bufs[0], gsems[0])
    for ch in range(NGCH):
        cur = ch % 2
        gathers[ch].wait()
        writes[ch] = pltpu.async_copy(
            bufs[cur], out_hbm.at[pl.ds(base + ch * GCHUNK, GCHUNK)],
            wsems[cur])
        if ch + 1 < NGCH:
            nxt = (ch + 1) % 2
            if ch >= 1:
                writes[ch - 1].wait()
            gathers[ch + 1] = pltpu.async_copy(
                emb_hbm.at[idx_v.at[pl.ds((ch + 1) * GCHUNK, GCHUNK)]],
                bufs[nxt], gsems[nxt])
    writes[NGCH - 1].wait()


def _sc_embed_gather(tok, embedding):
    mesh = plsc.VectorSubcoreMesh(core_axis_name="c", subcore_axis_name="s")
    f = functools.partial(
        pl.kernel,
        mesh=mesh,
        out_type=jax.ShapeDtypeStruct((NCROWS, D), _f32),
        scratch_types=[
            pltpu.VMEM((GPW,), _i32),
            pltpu.VMEM((GCHUNK, D), _f32),
            pltpu.VMEM((GCHUNK, D), _f32),
            pltpu.SemaphoreType.DMA,
            pltpu.SemaphoreType.DMA,
            pltpu.SemaphoreType.DMA,
            pltpu.SemaphoreType.DMA,
        ],
    )(_sc_embed_body)
    return f(tok, embedding)


def _prologue_body(cb_ref, cbn_ref, tbl_ref, g_ref):
    cb = cb_ref[...]
    n2 = jnp.sum(cb * cb, axis=1, keepdims=True)
    n = jnp.sqrt(n2)
    mcol = jnp.maximum(n, 1e-12)
    cbn_ref[...] = (cb / mcol).astype(jnp.bfloat16)
    pn = cb / (n + 1e-6)
    p2 = pn * pn
    s2 = jnp.sum(p2, axis=1, keepdims=True)
    s4 = jnp.sum(p2 * p2, axis=1, keepdims=True)
    s6 = jnp.sum(p2 * p2 * p2, axis=1, keepdims=True)
    z = jnp.zeros_like(s2)
    tbl_ref[...] = jnp.concatenate(
        [s2.T, s4.T, s6.T, n2.T, mcol.T, z.T, z.T, z.T], axis=0)
    g_ref[...] = lax.dot_general(
        pn, pn, (((1,), (1,)), ((), ())),
        preferred_element_type=_f32, precision=lax.Precision.HIGHEST)


def _prologue(codebook):
    return pl.pallas_call(
        _prologue_body,
        out_shape=[
            jax.ShapeDtypeStruct((K, D), jnp.bfloat16),
            jax.ShapeDtypeStruct((8, K), _f32),
            jax.ShapeDtypeStruct((K, K), _f32),
        ],
    )(codebook)


def _main_body(ze_ref, cbn_ref, k_ref, rmm_ref, cs_ref, rn_ref):
    i = pl.program_id(0)
    ze = ze_ref[...]
    rown2 = jnp.sum(ze * ze, axis=1, keepdims=True)
    m_row = jnp.maximum(jnp.sqrt(rown2), 1e-12)
    fn = (ze / m_row).astype(jnp.bfloat16)
    sim = lax.dot_general(
        fn, cbn_ref[...], (((1,), (1,)), ((), ())),
        preferred_element_type=_f32)
    rowmax = jnp.max(sim, axis=1, keepdims=True)
    io = lax.broadcasted_iota(_i32, sim.shape, 1)
    kk = jnp.min(jnp.where(sim == rowmax, io, K), axis=1)
    k_ref[...] = kk
    rmm_ref[...] = (rowmax * m_row)[:, 0]
    p = jnp.exp(sim - rowmax)
    probs = p * (1.0 / jnp.sum(p, axis=1, keepdims=True))

    @pl.when(i == 0)
    def _init():
        cs_ref[...] = jnp.zeros_like(cs_ref)
        rn_ref[...] = jnp.zeros_like(rn_ref)

    cs_ref[...] += jnp.sum(probs, axis=0, keepdims=True)
    rn_ref[...] += jnp.reshape(jnp.sum(rown2), (1, 1))


def _main(ze, cbn):
    return pl.pallas_call(
        _main_body,
        grid=(N_BLOCKS,),
        in_specs=[
            pl.BlockSpec((ROW_BLOCK, D), lambda i: (i, 0)),
            pl.BlockSpec((K, D), lambda i: (0, 0)),
        ],
        out_specs=[
            pl.BlockSpec((ROW_BLOCK,), lambda i: (i,)),
            pl.BlockSpec((ROW_BLOCK,), lambda i: (i,)),
            pl.BlockSpec((1, K), lambda i: (0, 0)),
            pl.BlockSpec((1, 1), lambda i: (0, 0)),
        ],
        out_shape=[
            jax.ShapeDtypeStruct((NCROWS,), _i32),
            jax.ShapeDtypeStruct((NCROWS,), _f32),
            jax.ShapeDtypeStruct((1, K), _f32),
            jax.ShapeDtypeStruct((1, 1), _f32),
        ],
    )(ze, cbn)


def _sc_tab_body(k_hbm, tbl_hbm, g_hbm,
                 s2o, s4o, s6o, c2o, mco, gno,
                 krow, tblv, gidx, s2b, s4b, s6b, c2b, mcb, gob, sem):
    c = lax.axis_index("c")
    s = lax.axis_index("s")
    wid = s * NC + c
    bl = wid // WPR
    t0 = (wid % WPR) * TPW
    pltpu.sync_copy(k_hbm.at[pl.ds(bl * T, T)], krow)
    pltpu.sync_copy(tbl_hbm, tblv)
    for j in range(TPW // 16):
        t = t0 + j * 16 + lax.iota(_i32, 16)
        iself = plsc.load_gather(krow, [t])
        inext = plsc.load_gather(krow, [lax.rem(t + 1, T)])
        sl = pl.ds(j * 16, 16)
        gidx[sl] = iself * K + inext
        s2b[sl] = plsc.load_gather(tblv, [iself])
        s4b[sl] = plsc.load_gather(tblv, [iself + K])
        s6b[sl] = plsc.load_gather(tblv, [iself + 2 * K])
        c2b[sl] = plsc.load_gather(tblv, [iself + 3 * K])
        mcb[sl] = plsc.load_gather(tblv, [iself + 4 * K])
    pltpu.async_copy(g_hbm.at[gidx], gob, sem).wait()
    base = wid * TPW
    pltpu.sync_copy(s2b, s2o.at[pl.ds(base, TPW)])
    pltpu.sync_copy(s4b, s4o.at[pl.ds(base, TPW)])
    pltpu.sync_copy(s6b, s6o.at[pl.ds(base, TPW)])
    pltpu.sync_copy(c2b, c2o.at[pl.ds(base, TPW)])
    pltpu.sync_copy(mcb, mco.at[pl.ds(base, TPW)])
    pltpu.sync_copy(gob, gno.at[pl.ds(base, TPW)])


def _sc_tab_gather(kchunk, tblflat, gflat):
    mesh = plsc.VectorSubcoreMesh(core_axis_name="c", subcore_axis_name="s")
    vecN = jax.ShapeDtypeStruct((NCROWS,), _f32)
    f = functools.partial(
        pl.kernel,
        mesh=mesh,
        compiler_params=pltpu.CompilerParams(needs_layout_passes=False),
        out_type=[vecN] * 6,
        scratch_types=[
            pltpu.VMEM((T,), _i32),
            pltpu.VMEM((8 * K,), _f32),
            pltpu.VMEM((TPW,), _i32),
            pltpu.VMEM((TPW,), _f32),
            pltpu.VMEM((TPW,), _f32),
            pltpu.VMEM((TPW,), _f32),
            pltpu.VMEM((TPW,), _f32),
            pltpu.VMEM((TPW,), _f32),
            pltpu.VMEM((TPW,), _f32),
            pltpu.SemaphoreType.DMA,
        ],
    )(_sc_tab_body)
    return f(kchunk, tblflat, gflat)


def _final_body(*refs):
    (s2c, s4c, s6c, c2c, mcc, gnc, rmc) = (
        refs[i * CH:(i + 1) * CH] for i in range(7))
    csc = refs[7 * CH:8 * CH]
    rnc = refs[8 * CH:9 * CH]
    lm2, lg4, lg6, mu_ = refs[9 * CH:9 * CH + 4]
    sd_ref, sm_ref, cl_ref, pp_ref = refs[9 * CH + 4:]
    m2 = jnp.exp(lm2[0, 0])
    g4 = jnp.exp(lg4[0, 0])
    g6 = jnp.exp(lg6[0, 0])
    emu = jnp.exp(mu_[0, 0])
    enmu = jnp.exp(-mu_[0, 0])
    cat = lambda rs: jnp.concatenate([r[...] for r in rs])
    s2 = cat(s2c)
    s4 = cat(s4c)
    s6 = cat(s6c)
    c2 = cat(c2c)
    mc = cat(mcc)
    gn = cat(gnc)
    rm = cat(rmc)
    tmod = lax.rem(lax.broadcasted_iota(_i32, (N,), 0), T)
    gp = jnp.where(tmod == 0, jnp.roll(gn, 1 - T), jnp.roll(gn, 1))
    chem = -0.5 * (emu * gn + enmu * gp)
    mass = (-0.5 * m2) * s2
    p4 = (-g4 / 24.0) * s4
    p6 = (-g6 / 720.0) * s6
    s2n = jnp.roll(s2, -1)
    kin = 0.5 * (s2 + s2n) - gn
    sd = mass + p4 + p6 + chem + jnp.where(tmod < T - 1, kin, 0.0)
    sd_ref[...] = sd
    sm_ref[...] = jnp.reshape(
        jnp.stack([jnp.sum(sd[b * T:(b + 1) * T]) for b in range(B)]), (1, B))
    rntot = sum(r[0, 0] for r in rnc)
    commit = (rntot - 2.0 * jnp.sum(rm * mc)
              + jnp.sum(c2)) * (1.0 / (N * D))
    cl_ref[...] = jnp.reshape(commit, (1, 1))
    avg = sum(c[...] for c in csc) * (1.0 / N)
    pp = jnp.exp(-jnp.sum(avg * jnp.log(avg + 1e-10)))
    pp_ref[...] = jnp.reshape(pp, (1, 1))


def _final(args):
    return pl.pallas_call(
        _final_body,
        out_shape=[
            jax.ShapeDtypeStruct((N,), _f32),
            jax.ShapeDtypeStruct((1, B), _f32),
            jax.ShapeDtypeStruct((1, 1), _f32),
            jax.ShapeDtypeStruct((1, 1), _f32),
        ],
    )(*args)


def kernel(token_ids, embedding, codebook, log_m2, log_g4, log_g6, mu):
    tok = token_ids.reshape(-1).astype(_i32)
    zes = [_sc_embed_gather(tok[c * NCROWS:(c + 1) * NCROWS], embedding)
           for c in range(CH)]
    cbn, tbl, g = _prologue(codebook)
    tblflat = tbl.reshape(-1)
    gflat = g.reshape(-1)
    mains = [_main(ze, cbn) for ze in zes]
    tabs = [_sc_tab_gather(m[0], tblflat, gflat) for m in mains]
    p11 = lambda x: jnp.reshape(x.astype(_f32), (1, 1))
    args = []
    for i in range(6):
        args += [t[i] for t in tabs]
    args += [m[1] for m in mains]
    args += [m[2] for m in mains]
    args += [m[3] for m in mains]
    args += [p11(log_m2), p11(log_g4), p11(log_g6), p11(mu)]
    sd, sm, cl, pp = _final(args)
    quanta = jnp.concatenate([m[0] for m in mains]).reshape(B, T)
    return (sm.reshape(B), sd.reshape(B, T), quanta, cl.reshape(()),
            pp.reshape(()))


# speedup vs baseline: 1.2015x; 1.0157x over previous
import functools

import jax
import jax.numpy as jnp
from jax import lax
from jax.experimental import pallas as pl
from jax.experimental.pallas import tpu as pltpu
from jax.experimental.pallas import tpu_sc as plsc

B, T = 4, 2048
D, K = 1024, 512
N = B * T
NC, NS = 2, 16
NW = NC * NS
CH = 2
NCROWS = N // CH
GPW = NCROWS // NW
GCHUNK = 32
NGCH = GPW // GCHUNK
TPW = NCROWS // NW
WPR = T // TPW
ROW_BLOCK = 1024
N_BLOCKS = NCROWS // ROW_BLOCK

_f32 = jnp.float32
_i32 = jnp.int32


def _sc_embed_body(tok_hbm, emb_hbm, out_hbm, idx_v, buf_a, buf_b,
                   gsem_a, gsem_b, wsem_a, wsem_b):
    c = lax.axis_index("c")
    s = lax.axis_index("s")
    wid = s * NC + c
    base = wid * GPW
    pltpu.sync_copy(tok_hbm.at[pl.ds(base, GPW)], idx_v)
    bufs = (buf_a, buf_b)
    gsems = (gsem_a, gsem_b)
    wsems = (wsem_a, wsem_b)
    gathers = [None] * NGCH
    writes = [None] * NGCH
    gathers[0] = pltpu.async_copy(
        emb_hbm.at[idx_v.at[pl.ds(0, GCHUNK)]], bufs[0], gsems[0])
    for ch in range(NGCH):
        cur = ch % 2
        gathers[ch].wait()
        writes[ch] = pltpu.async_copy(
            bufs[cur], out_hbm.at[pl.ds(base + ch * GCHUNK, GCHUNK)],
            wsems[cur])
        if ch + 1 < NGCH:
            nxt = (ch + 1) % 2
            if ch >= 1:
                writes[ch - 1].wait()
            gathers[ch + 1] = pltpu.async_copy(
                emb_hbm.at[idx_v.at[pl.ds((ch + 1) * GCHUNK, GCHUNK)]],
                bufs[nxt], gsems[nxt])
    writes[NGCH - 1].wait()


def _sc_embed_gather(tok, embedding):
    mesh = plsc.VectorSubcoreMesh(core_axis_name="c", subcore_axis_name="s")
    f = functools.partial(
        pl.kernel,
        mesh=mesh,
        out_type=jax.ShapeDtypeStruct((NCROWS, D), _f32),
        scratch_types=[
            pltpu.VMEM((GPW,), _i32),
            pltpu.VMEM((GCHUNK, D), _f32),
            pltpu.VMEM((GCHUNK, D), _f32),
            pltpu.SemaphoreType.DMA,
            pltpu.SemaphoreType.DMA,
            pltpu.SemaphoreType.DMA,
            pltpu.SemaphoreType.DMA,
        ],
    )(_sc_embed_body)
    return f(tok, embedding)


def _prologue_body(cb_ref, cbn_ref, tbl_ref, g_ref):
    cb = cb_ref[...]
    n2 = jnp.sum(cb * cb, axis=1, keepdims=True)
    n = jnp.sqrt(n2)
    mcol = jnp.maximum(n, 1e-12)
    cbn_ref[...] = (cb / mcol).astype(jnp.bfloat16)
    pn = cb / (n + 1e-6)
    p2 = pn * pn
    s2 = jnp.sum(p2, axis=1, keepdims=True)
    s4 = jnp.sum(p2 * p2, axis=1, keepdims=True)
    s6 = jnp.sum(p2 * p2 * p2, axis=1, keepdims=True)
    z = jnp.zeros_like(s2)
    tbl_ref[...] = jnp.concatenate(
        [s2.T, s4.T, s6.T, n2.T, mcol.T, z.T, z.T, z.T], axis=0)
    g_ref[...] = lax.dot_general(
        pn, pn, (((1,), (1,)), ((), ())),
        preferred_element_type=_f32, precision=lax.Precision.HIGHEST)


def _prologue(codebook):
    return pl.pallas_call(
        _prologue_body,
        out_shape=[
            jax.ShapeDtypeStruct((K, D), jnp.bfloat16),
            jax.ShapeDtypeStruct((8, K), _f32),
            jax.ShapeDtypeStruct((K, K), _f32),
        ],
    )(codebook)


def _main_body(ze_ref, cbn_ref, k_ref, rmm_ref, cs_ref, rn_ref):
    i = pl.program_id(0)
    ze = ze_ref[...]
    rown2 = jnp.sum(ze * ze, axis=1, keepdims=True)
    m_row = jnp.maximum(jnp.sqrt(rown2), 1e-12)
    fn = (ze / m_row).astype(jnp.bfloat16)
    sim = lax.dot_general(
        fn, cbn_ref[...], (((1,), (1,)), ((), ())),
        preferred_element_type=_f32)
    rowmax = jnp.max(sim, axis=1, keepdims=True)
    io = lax.broadcasted_iota(_i32, sim.shape, 1)
    kk = jnp.min(jnp.where(sim == rowmax, io, K), axis=1)
    k_ref[...] = kk
    rmm_ref[...] = (rowmax * m_row)[:, 0]
    p = jnp.exp(sim - rowmax)
    probs = p * (1.0 / jnp.sum(p, axis=1, keepdims=True))

    @pl.when(i == 0)
    def _init():
        cs_ref[...] = jnp.zeros_like(cs_ref)
        rn_ref[...] = jnp.zeros_like(rn_ref)

    cs_ref[...] += jnp.sum(probs, axis=0, keepdims=True)
    rn_ref[...] += jnp.reshape(jnp.sum(rown2), (1, 1))


def _main(ze, cbn):
    return pl.pallas_call(
        _main_body,
        grid=(N_BLOCKS,),
        in_specs=[
            pl.BlockSpec((ROW_BLOCK, D), lambda i: (i, 0)),
            pl.BlockSpec((K, D), lambda i: (0, 0)),
        ],
        out_specs=[
            pl.BlockSpec((ROW_BLOCK,), lambda i: (i,)),
            pl.BlockSpec((ROW_BLOCK,), lambda i: (i,)),
            pl.BlockSpec((1, K), lambda i: (0, 0)),
            pl.BlockSpec((1, 1), lambda i: (0, 0)),
        ],
        out_shape=[
            jax.ShapeDtypeStruct((NCROWS,), _i32),
            jax.ShapeDtypeStruct((NCROWS,), _f32),
            jax.ShapeDtypeStruct((1, K), _f32),
            jax.ShapeDtypeStruct((1, 1), _f32),
        ],
    )(ze, cbn)


def _sc_tab_body(k_hbm, tbl_hbm, g_hbm,
                 s2o, s4o, s6o, c2o, mco, gno,
                 krow, tblv, gidx, s2b, s4b, s6b, c2b, mcb, gob, sem):
    c = lax.axis_index("c")
    s = lax.axis_index("s")
    wid = s * NC + c
    bl = wid // WPR
    t0 = (wid % WPR) * TPW
    pltpu.sync_copy(k_hbm.at[pl.ds(bl * T, T)], krow)
    pltpu.sync_copy(tbl_hbm, tblv)
    for j in range(TPW // 16):
        t = t0 + j * 16 + lax.iota(_i32, 16)
        iself = plsc.load_gather(krow, [t])
        inext = plsc.load_gather(krow, [lax.rem(t + 1, T)])
        sl = pl.ds(j * 16, 16)
        gidx[sl] = iself * K + inext
        s2b[sl] = plsc.load_gather(tblv, [iself])
        s4b[sl] = plsc.load_gather(tblv, [iself + K])
        s6b[sl] = plsc.load_gather(tblv, [iself + 2 * K])
        c2b[sl] = plsc.load_gather(tblv, [iself + 3 * K])
        mcb[sl] = plsc.load_gather(tblv, [iself + 4 * K])
    pltpu.async_copy(g_hbm.at[gidx], gob, sem).wait()
    base = wid * TPW
    pltpu.sync_copy(s2b, s2o.at[pl.ds(base, TPW)])
    pltpu.sync_copy(s4b, s4o.at[pl.ds(base, TPW)])
    pltpu.sync_copy(s6b, s6o.at[pl.ds(base, TPW)])
    pltpu.sync_copy(c2b, c2o.at[pl.ds(base, TPW)])
    pltpu.sync_copy(mcb, mco.at[pl.ds(base, TPW)])
    pltpu.sync_copy(gob, gno.at[pl.ds(base, TPW)])


def _sc_tab_gather(kchunk, tblflat, gflat):
    mesh = plsc.VectorSubcoreMesh(core_axis_name="c", subcore_axis_name="s")
    vecN = jax.ShapeDtypeStruct((NCROWS,), _f32)
    f = functools.partial(
        pl.kernel,
        mesh=mesh,
        compiler_params=pltpu.CompilerParams(needs_layout_passes=False),
        out_type=[vecN] * 6,
        scratch_types=[
            pltpu.VMEM((T,), _i32),
            pltpu.VMEM((8 * K,), _f32),
            pltpu.VMEM((TPW,), _i32),
            pltpu.VMEM((TPW,), _f32),
            pltpu.VMEM((TPW,), _f32),
            pltpu.VMEM((TPW,), _f32),
            pltpu.VMEM((TPW,), _f32),
            pltpu.VMEM((TPW,), _f32),
            pltpu.VMEM((TPW,), _f32),
            pltpu.SemaphoreType.DMA,
        ],
    )(_sc_tab_body)
    return f(kchunk, tblflat, gflat)


def _final_body(*refs):
    (s2c, s4c, s6c, c2c, mcc, gnc, rmc, kc) = (
        refs[i * CH:(i + 1) * CH] for i in range(8))
    csc = refs[8 * CH:9 * CH]
    rnc = refs[9 * CH:10 * CH]
    lm2, lg4, lg6, mu_ = refs[10 * CH:10 * CH + 4]
    sd_ref, q_ref, sm_ref, cl_ref, pp_ref = refs[10 * CH + 4:]
    m2 = jnp.exp(lm2[0, 0])
    g4 = jnp.exp(lg4[0, 0])
    g6 = jnp.exp(lg6[0, 0])
    emu = jnp.exp(mu_[0, 0])
    enmu = jnp.exp(-mu_[0, 0])
    cat = lambda rs: jnp.concatenate([r[...] for r in rs])
    s2 = cat(s2c)
    s4 = cat(s4c)
    s6 = cat(s6c)
    c2 = cat(c2c)
    mc = cat(mcc)
    gn = cat(gnc)
    rm = cat(rmc)
    tmod = lax.rem(lax.broadcasted_iota(_i32, (N,), 0), T)
    gp = jnp.where(tmod == 0, jnp.roll(gn, 1 - T), jnp.roll(gn, 1))
    chem = -0.5 * (emu * gn + enmu * gp)
    mass = (-0.5 * m2) * s2
    p4 = (-g4 / 24.0) * s4
    p6 = (-g6 / 720.0) * s6
    s2n = jnp.roll(s2, -1)
    kin = 0.5 * (s2 + s2n) - gn
    sd = mass + p4 + p6 + chem + jnp.where(tmod < T - 1, kin, 0.0)
    sd_ref[...] = jnp.reshape(sd, (B, T))
    q_ref[...] = jnp.reshape(cat(kc), (B, T))
    sm_ref[...] = jnp.reshape(
        jnp.stack([jnp.sum(sd[b * T:(b + 1) * T]) for b in range(B)]), (1, B))
    rntot = sum(r[0, 0] for r in rnc)
    commit = (rntot - 2.0 * jnp.sum(rm * mc)
              + jnp.sum(c2)) * (1.0 / (N * D))
    cl_ref[...] = jnp.reshape(commit, (1, 1))
    avg = sum(c[...] for c in csc) * (1.0 / N)
    pp = jnp.exp(-jnp.sum(avg * jnp.log(avg + 1e-10)))
    pp_ref[...] = jnp.reshape(pp, (1, 1))


def _final(args):
    return pl.pallas_call(
        _final_body,
        out_shape=[
            jax.ShapeDtypeStruct((B, T), _f32),
            jax.ShapeDtypeStruct((B, T), _i32),
            jax.ShapeDtypeStruct((1, B), _f32),
            jax.ShapeDtypeStruct((1, 1), _f32),
            jax.ShapeDtypeStruct((1, 1), _f32),
        ],
    )(*args)


def kernel(token_ids, embedding, codebook, log_m2, log_g4, log_g6, mu):
    tok = token_ids.reshape(-1).astype(_i32)
    zes = [_sc_embed_gather(tok[c * NCROWS:(c + 1) * NCROWS], embedding)
           for c in range(CH)]
    cbn, tbl, g = _prologue(codebook)
    tblflat = tbl.reshape(-1)
    gflat = g.reshape(-1)
    mains = [_main(ze, cbn) for ze in zes]
    tabs = [_sc_tab_gather(m[0], tblflat, gflat) for m in mains]
    p11 = lambda x: jnp.reshape(x.astype(_f32), (1, 1))
    args = []
    for i in range(6):
        args += [t[i] for t in tabs]
    args += [m[1] for m in mains]
    args += [m[0] for m in mains]
    args += [m[2] for m in mains]
    args += [m[3] for m in mains]
    args += [p11(log_m2), p11(log_g4), p11(log_g6), p11(mu)]
    sd, quanta, sm, cl, pp = _final(args)
    return (sm.reshape(B), sd, quanta, cl.reshape(()), pp.reshape(()))


# speedup vs baseline: 1.2358x; 1.0286x over previous
import functools

import jax
import jax.numpy as jnp
from jax import lax
from jax.experimental import pallas as pl
from jax.experimental.pallas import tpu as pltpu
from jax.experimental.pallas import tpu_sc as plsc

B, T = 4, 2048
D, K = 1024, 512
N = B * T
NC, NS = 2, 16
NW = NC * NS
CH = 2
NCROWS = N // CH
GPW = NCROWS // NW
GCHUNK = 32
NGCH = GPW // GCHUNK
TPW = NCROWS // NW
WPR = T // TPW
ROW_BLOCK = 1024
N_BLOCKS = NCROWS // ROW_BLOCK

_f32 = jnp.float32
_i32 = jnp.int32


def _sc_embed_body(tok_hbm, emb_hbm, out_hbm, idx_v, buf_a, buf_b,
                   gsem_a, gsem_b, wsem_a, wsem_b):
    c = lax.axis_index("c")
    s = lax.axis_index("s")
    wid = s * NC + c
    base = wid * GPW
    pltpu.sync_copy(tok_hbm.at[pl.ds(base, GPW)], idx_v)
    bufs = (buf_a, buf_b)
    gsems = (gsem_a, gsem_b)
    wsems = (wsem_a, wsem_b)
    gathers = [None] * NGCH
    writes = [None] * NGCH
    gathers[0] = pltpu.async_copy(
        emb_hbm.at[idx_v.at[pl.ds(0, GCHUNK)]], bufs[0], gsems[0])
    for ch in range(NGCH):
        cur = ch % 2
        gathers[ch].wait()
        writes[ch] = pltpu.async_copy(
            bufs[cur], out_hbm.at[pl.ds(base + ch * GCHUNK, GCHUNK)],
            wsems[cur])
        if ch + 1 < NGCH:
            nxt = (ch + 1) % 2
            if ch >= 1:
                writes[ch - 1].wait()
            gathers[ch + 1] = pltpu.async_copy(
                emb_hbm.at[idx_v.at[pl.ds((ch + 1) * GCHUNK, GCHUNK)]],
                bufs[nxt], gsems[nxt])
    writes[NGCH - 1].wait()


def _sc_embed_gather(tok, embedding):
    mesh = plsc.VectorSubcoreMesh(core_axis_name="c", subcore_axis_name="s")
    f = functools.partial(
        pl.kernel,
        mesh=mesh,
        out_type=jax.ShapeDtypeStruct((NCROWS, D), _f32),
        scratch_types=[
            pltpu.VMEM((GPW,), _i32),
            pltpu.VMEM((GCHUNK, D), _f32),
            pltpu.VMEM((GCHUNK, D), _f32),
            pltpu.SemaphoreType.DMA,
            pltpu.SemaphoreType.DMA,
            pltpu.SemaphoreType.DMA,
            pltpu.SemaphoreType.DMA,
        ],
    )(_sc_embed_body)
    return f(tok, embedding)


def _prologue_body(cb_ref, cbn_ref, tbl_ref, g_ref):
    cb = cb_ref[...]
    n2 = jnp.sum(cb * cb, axis=1, keepdims=True)
    n = jnp.sqrt(n2)
    mcol = jnp.maximum(n, 1e-12)
    cbn_ref[...] = (cb / mcol).astype(jnp.bfloat16)
    pn = cb / (n + 1e-6)
    p2 = pn * pn
    s2 = jnp.sum(p2, axis=1, keepdims=True)
    s4 = jnp.sum(p2 * p2, axis=1, keepdims=True)
    s6 = jnp.sum(p2 * p2 * p2, axis=1, keepdims=True)
    z = jnp.zeros_like(s2)
    tbl_ref[...] = jnp.concatenate(
        [s2.T, s4.T, s6.T, n2.T, mcol.T, z.T, z.T, z.T], axis=0)
    g_ref[...] = lax.dot_general(
        pn, pn, (((1,), (1,)), ((), ())),
        preferred_element_type=_f32, precision=lax.Precision.HIGHEST)


def _prologue(codebook):
    return pl.pallas_call(
        _prologue_body,
        out_shape=[
            jax.ShapeDtypeStruct((K, D), jnp.bfloat16),
            jax.ShapeDtypeStruct((8, K), _f32),
            jax.ShapeDtypeStruct((K, K), _f32),
        ],
    )(codebook)


def _main_body(ze_ref, cbn_ref, k_ref, rmm_ref, cs_ref, rn_ref):
    i = pl.program_id(0)
    ze = ze_ref[...]
    rown2 = jnp.sum(ze * ze, axis=1, keepdims=True)
    m_row = jnp.maximum(jnp.sqrt(rown2), 1e-12)
    fn = (ze / m_row).astype(jnp.bfloat16)
    sim = lax.dot_general(
        fn, cbn_ref[...], (((1,), (1,)), ((), ())),
        preferred_element_type=_f32)
    rowmax = jnp.max(sim, axis=1, keepdims=True)
    io = lax.broadcasted_iota(_i32, sim.shape, 1)
    kk = jnp.min(jnp.where(sim == rowmax, io, K), axis=1)
    k_ref[...] = kk
    rmm_ref[...] = (rowmax * m_row)[:, 0]
    p = jnp.exp(sim - rowmax)
    psum = lax.dot_general(
        p.astype(jnp.bfloat16), jnp.ones((K, 128), jnp.bfloat16),
        (((1,), (0,)), ((), ())), preferred_element_type=_f32)
    probs = p * (1.0 / psum[:, :1])

    @pl.when(i == 0)
    def _init():
        cs_ref[...] = jnp.zeros_like(cs_ref)
        rn_ref[...] = jnp.zeros_like(rn_ref)

    cs_ref[...] += jnp.sum(probs, axis=0, keepdims=True)
    rn_ref[...] += jnp.reshape(jnp.sum(rown2), (1, 1))


def _main(ze, cbn):
    return pl.pallas_call(
        _main_body,
        grid=(N_BLOCKS,),
        in_specs=[
            pl.BlockSpec((ROW_BLOCK, D), lambda i: (i, 0)),
            pl.BlockSpec((K, D), lambda i: (0, 0)),
        ],
        out_specs=[
            pl.BlockSpec((ROW_BLOCK,), lambda i: (i,)),
            pl.BlockSpec((ROW_BLOCK,), lambda i: (i,)),
            pl.BlockSpec((1, K), lambda i: (0, 0)),
            pl.BlockSpec((1, 1), lambda i: (0, 0)),
        ],
        out_shape=[
            jax.ShapeDtypeStruct((NCROWS,), _i32),
            jax.ShapeDtypeStruct((NCROWS,), _f32),
            jax.ShapeDtypeStruct((1, K), _f32),
            jax.ShapeDtypeStruct((1, 1), _f32),
        ],
    )(ze, cbn)


def _sc_tab_body(k_hbm, tbl_hbm, g_hbm,
                 s2o, s4o, s6o, c2o, mco, gno,
                 krow, tblv, gidx, s2b, s4b, s6b, c2b, mcb, gob, sem):
    c = lax.axis_index("c")
    s = lax.axis_index("s")
    wid = s * NC + c
    bl = wid // WPR
    t0 = (wid % WPR) * TPW
    pltpu.sync_copy(k_hbm.at[pl.ds(bl * T, T)], krow)
    pltpu.sync_copy(tbl_hbm, tblv)
    for j in range(TPW // 16):
        t = t0 + j * 16 + lax.iota(_i32, 16)
        iself = plsc.load_gather(krow, [t])
        inext = plsc.load_gather(krow, [lax.rem(t + 1, T)])
        sl = pl.ds(j * 16, 16)
        gidx[sl] = iself * K + inext
        s2b[sl] = plsc.load_gather(tblv, [iself])
        s4b[sl] = plsc.load_gather(tblv, [iself + K])
        s6b[sl] = plsc.load_gather(tblv, [iself + 2 * K])
        c2b[sl] = plsc.load_gather(tblv, [iself + 3 * K])
        mcb[sl] = plsc.load_gather(tblv, [iself + 4 * K])
    pltpu.async_copy(g_hbm.at[gidx], gob, sem).wait()
    base = wid * TPW
    pltpu.sync_copy(s2b, s2o.at[pl.ds(base, TPW)])
    pltpu.sync_copy(s4b, s4o.at[pl.ds(base, TPW)])
    pltpu.sync_copy(s6b, s6o.at[pl.ds(base, TPW)])
    pltpu.sync_copy(c2b, c2o.at[pl.ds(base, TPW)])
    pltpu.sync_copy(mcb, mco.at[pl.ds(base, TPW)])
    pltpu.sync_copy(gob, gno.at[pl.ds(base, TPW)])


def _sc_tab_gather(kchunk, tblflat, gflat):
    mesh = plsc.VectorSubcoreMesh(core_axis_name="c", subcore_axis_name="s")
    vecN = jax.ShapeDtypeStruct((NCROWS,), _f32)
    f = functools.partial(
        pl.kernel,
        mesh=mesh,
        compiler_params=pltpu.CompilerParams(needs_layout_passes=False),
        out_type=[vecN] * 6,
        scratch_types=[
            pltpu.VMEM((T,), _i32),
            pltpu.VMEM((8 * K,), _f32),
            pltpu.VMEM((TPW,), _i32),
            pltpu.VMEM((TPW,), _f32),
            pltpu.VMEM((TPW,), _f32),
            pltpu.VMEM((TPW,), _f32),
            pltpu.VMEM((TPW,), _f32),
            pltpu.VMEM((TPW,), _f32),
            pltpu.VMEM((TPW,), _f32),
            pltpu.SemaphoreType.DMA,
        ],
    )(_sc_tab_body)
    return f(kchunk, tblflat, gflat)


def _final_body(*refs):
    (s2c, s4c, s6c, c2c, mcc, gnc, rmc, kc) = (
        refs[i * CH:(i + 1) * CH] for i in range(8))
    csc = refs[8 * CH:9 * CH]
    rnc = refs[9 * CH:10 * CH]
    lm2, lg4, lg6, mu_ = refs[10 * CH:10 * CH + 4]
    sd_ref, q_ref, sm_ref, cl_ref, pp_ref = refs[10 * CH + 4:]
    m2 = jnp.exp(lm2[0, 0])
    g4 = jnp.exp(lg4[0, 0])
    g6 = jnp.exp(lg6[0, 0])
    emu = jnp.exp(mu_[0, 0])
    enmu = jnp.exp(-mu_[0, 0])
    cat = lambda rs: jnp.concatenate([r[...] for r in rs])
    s2 = cat(s2c)
    s4 = cat(s4c)
    s6 = cat(s6c)
    c2 = cat(c2c)
    mc = cat(mcc)
    gn = cat(gnc)
    rm = cat(rmc)
    tmod = lax.rem(lax.broadcasted_iota(_i32, (N,), 0), T)
    gp = jnp.where(tmod == 0, jnp.roll(gn, 1 - T), jnp.roll(gn, 1))
    chem = -0.5 * (emu * gn + enmu * gp)
    mass = (-0.5 * m2) * s2
    p4 = (-g4 / 24.0) * s4
    p6 = (-g6 / 720.0) * s6
    s2n = jnp.roll(s2, -1)
    kin = 0.5 * (s2 + s2n) - gn
    sd = mass + p4 + p6 + chem + jnp.where(tmod < T - 1, kin, 0.0)
    sd_ref[...] = jnp.reshape(sd, (B, T))
    q_ref[...] = jnp.reshape(cat(kc), (B, T))
    sm_ref[...] = jnp.reshape(
        jnp.stack([jnp.sum(sd[b * T:(b + 1) * T]) for b in range(B)]), (1, B))
    rntot = sum(r[0, 0] for r in rnc)
    commit = (rntot - 2.0 * jnp.sum(rm * mc)
              + jnp.sum(c2)) * (1.0 / (N * D))
    cl_ref[...] = jnp.reshape(commit, (1, 1))
    avg = sum(c[...] for c in csc) * (1.0 / N)
    pp = jnp.exp(-jnp.sum(avg * jnp.log(avg + 1e-10)))
    pp_ref[...] = jnp.reshape(pp, (1, 1))


def _final(args):
    return pl.pallas_call(
        _final_body,
        out_shape=[
            jax.ShapeDtypeStruct((B, T), _f32),
            jax.ShapeDtypeStruct((B, T), _i32),
            jax.ShapeDtypeStruct((1, B), _f32),
            jax.ShapeDtypeStruct((1, 1), _f32),
            jax.ShapeDtypeStruct((1, 1), _f32),
        ],
    )(*args)


def kernel(token_ids, embedding, codebook, log_m2, log_g4, log_g6, mu):
    tok = token_ids.reshape(-1).astype(_i32)
    zes = [_sc_embed_gather(tok[c * NCROWS:(c + 1) * NCROWS], embedding)
           for c in range(CH)]
    cbn, tbl, g = _prologue(codebook)
    tblflat = tbl.reshape(-1)
    gflat = g.reshape(-1)
    mains = [_main(ze, cbn) for ze in zes]
    tabs = [_sc_tab_gather(m[0], tblflat, gflat) for m in mains]
    p11 = lambda x: jnp.reshape(x.astype(_f32), (1, 1))
    args = []
    for i in range(6):
        args += [t[i] for t in tabs]
    args += [m[1] for m in mains]
    args += [m[0] for m in mains]
    args += [m[2] for m in mains]
    args += [m[3] for m in mains]
    args += [p11(log_m2), p11(log_g4), p11(log_g6), p11(mu)]
    sd, quanta, sm, cl, pp = _final(args)
    return (sm.reshape(B), sd, quanta, cl.reshape(()), pp.reshape(()))
